# baseline (speedup 1.0000x reference)
import functools

import jax
import jax.numpy as jnp
from jax import lax
from jax.experimental import pallas as pl
from jax.experimental.pallas import tpu as pltpu
from jax.experimental.pallas import tpu_sc as plsc

_NC = 2
_NS = 16
_NW = _NC * _NS
_C = 128


def _emb_body(idx_hbm, table_hbm, pos2_hbm, out_hbm,
              idx_v, pos_v, rows_a, rows_b, sem_a, sem_b,
              *, D, L, nchunk):
    wid = lax.axis_index("s") * _NC + lax.axis_index("c")
    pltpu.sync_copy(idx_hbm.at[pl.ds(wid * nchunk, nchunk)], idx_v)
    pltpu.sync_copy(pos2_hbm, pos_v)

    row_base = wid * nchunk * _C

    def gather(chunk, rows, sem):
        pltpu.async_copy(table_hbm.at[idx_v.at[chunk]], rows, sem)

    def gwait(rows, sem):
        pltpu.make_async_copy(table_hbm.at[idx_v.at[0]], rows, sem).wait()

    def add_pos_and_flush(rows, chunk):
        off = lax.rem(chunk * _C, L)

        def body(i, carry):
            p = off + i
            for c in range(D // 16):
                sl = pl.ds(c * 16, 16)
                plsc.addupdate(rows.at[i, sl], pos_v[p, sl])
            return carry

        lax.fori_loop(0, _C, body, 0, unroll=4)
        pltpu.sync_copy(rows, out_hbm.at[pl.ds(row_base + chunk * _C, _C)])

    gather(0, rows_a, sem_a)

    kk_tot = nchunk // 2

    def step(kk, carry):
        k0 = 2 * kk
        gwait(rows_a, sem_a)
        gather(k0 + 1, rows_b, sem_b)
        add_pos_and_flush(rows_a, k0)
        gwait(rows_b, sem_b)

        @pl.when(kk < kk_tot - 1)
        def _():
            gather(k0 + 2, rows_a, sem_a)

        add_pos_and_flush(rows_b, k0 + 1)
        return carry

    lax.fori_loop(0, kk_tot, step, 0)


def kernel(inputs, token_table, pos_table):
    B, L = inputs.shape
    V, D = token_table.shape
    assert pos_table.shape == (L, D)
    total = B * L
    assert D % 16 == 0
    nchunk = total // (_NW * _C)
    assert total == _NW * nchunk * _C and nchunk % 2 == 0

    idx2d = inputs.reshape(_NW * nchunk, _C)
    if idx2d.dtype != jnp.int32:
        idx2d = idx2d.astype(jnp.int32)
    pos2 = jnp.concatenate([pos_table, pos_table], axis=0)

    mesh = plsc.VectorSubcoreMesh(core_axis_name="c", subcore_axis_name="s")
    run = pl.kernel(
        functools.partial(_emb_body, D=D, L=L, nchunk=nchunk),
        mesh=mesh,
        compiler_params=pltpu.CompilerParams(use_tc_tiling_on_sc=False),
        out_type=jax.ShapeDtypeStruct((total, D), jnp.float32),
        scratch_types=[
            pltpu.VMEM((nchunk, _C), jnp.int32),
            pltpu.VMEM((2 * L, D), jnp.float32),
            pltpu.VMEM((_C, D), jnp.float32),
            pltpu.VMEM((_C, D), jnp.float32),
            pltpu.SemaphoreType.DMA,
            pltpu.SemaphoreType.DMA,
        ],
    )
    out = run(idx2d, token_table, pos2)
    return out.reshape(B, L, D)
